# baseline (speedup 1.0000x reference)
import functools

import jax
import jax.numpy as jnp
from jax import lax
from jax.experimental import pallas as pl
from jax.experimental.pallas import tpu as pltpu
from jax.experimental.pallas import tpu_sc as plsc


def _retile_body(t1_ref, t2_ref, o_ref):
    z = jnp.concatenate([t1_ref[...], t2_ref[...]], axis=0)
    o_ref[...] = jnp.transpose(z)


def _retile(t, p, bc):
    n = t.shape[1]
    grid = p // bc
    max_blk = (n - 1) // bc

    return pl.pallas_call(
        _retile_body,
        grid=(grid,),
        in_specs=[
            pl.BlockSpec((64, bc), lambda i: (0, i)),
            pl.BlockSpec(
                (64, bc),
                lambda i, g=grid, m=max_blk: (0, jnp.minimum(i + g, m)),
            ),
        ],
        out_specs=pl.BlockSpec((bc, 128), lambda i: (i, 0)),
        out_shape=jax.ShapeDtypeStruct((p, 128), jnp.float32),
    )(t, t)


def _dynmap_sc(h2, hp_, t2, tp_, r2, rp_, ee, ep, rel_e, rel_p, d_real):
    B2 = h2.shape[0]
    D = 128
    info = plsc.get_sparse_core_info()
    NC, NS, L = info.num_cores, info.num_subcores, info.num_lanes
    NW = NC * NS
    C = 64
    b_per_w = B2 // NW
    n_chunks = b_per_w // C
    KV = d_real // L

    mesh = plsc.VectorSubcoreMesh(core_axis_name="c", subcore_axis_name="s")
    f32 = jnp.float32
    out_type = (
        jax.ShapeDtypeStruct((B2,), f32),
        jax.ShapeDtypeStruct((B2, D), f32),
        jax.ShapeDtypeStruct((B2, D), f32),
    )

    idx_t = pltpu.VMEM((C,), jnp.int32)
    row_t = pltpu.VMEM((C, D), f32)

    @functools.partial(
        pl.kernel,
        mesh=mesh,
        out_type=out_type,
        compiler_params=pltpu.CompilerParams(needs_layout_passes=False),
        scratch_types=[
            [[idx_t] * 6, [idx_t] * 6],
            [[row_t] * 6, [row_t] * 6],
            [pltpu.VMEM((C,), f32)] * 2,
            [pltpu.SemaphoreType.DMA] * 2,
        ],
    )
    def k(h2r, hpr, t2r, tpr, r2r, rpr, eet, ept, rele, relp,
          dist_o, h_out, t_out, idx_s, row_s, dv_s, sem_s):
        wid = lax.axis_index("s") * NC + lax.axis_index("c")
        lane = lax.iota(jnp.int32, L)
        idx_in = (h2r, t2r, r2r, hpr, tpr, rpr)

        def stage_and_fire(s, cc):
            base = wid * b_per_w + cc * C
            for j in range(6):
                pltpu.sync_copy(idx_in[j].at[pl.ds(base, C)], idx_s[s][j])
            hi, ti, ri = idx_s[s][0], idx_s[s][1], idx_s[s][2]
            he, hp, te, tp, re_v, rp_v = row_s[s]
            for cp in (
                pltpu.async_copy(eet.at[hi], he, sem_s[s]),
                pltpu.async_copy(ept.at[hi], hp, sem_s[s]),
                pltpu.async_copy(eet.at[ti], te, sem_s[s]),
                pltpu.async_copy(ept.at[ti], tp, sem_s[s]),
                pltpu.async_copy(rele.at[ri], re_v, sem_s[s]),
                pltpu.async_copy(relp.at[ri], rp_v, sem_s[s]),
            ):
                pass

        def drain(s):
            he, hp, te, tp, re_v, rp_v = row_s[s]
            for dst in (he, hp, te, tp, re_v, rp_v):
                pltpu.make_async_copy(eet.at[pl.ds(0, C)], dst,
                                      sem_s[s]).wait()

        def compute(s, cc):
            base = wid * b_per_w + cc * C
            he, hp, te, tp, re_v, rp_v = row_s[s]
            dv = dv_s[s]
            hoff_a, toff_a, roff_a = idx_s[s][3], idx_s[s][4], idx_s[s][5]

            def group_body(g, gcarry):
                d_vec = jnp.zeros((L,), f32)
                hoff_v = hoff_a[pl.ds(g * L, L)]
                toff_v = toff_a[pl.ds(g * L, L)]
                roff_v = roff_a[pl.ds(g * L, L)]
                for rl in range(L):
                    row = g * L + rl
                    hb = hoff_v[rl]
                    tb = toff_v[rl]
                    rb = roff_v[rl]
                    hv = [he[row, pl.ds(hb + kk * L, L)] for kk in range(KV)]
                    hpv = [hp[row, pl.ds(hb + kk * L, L)] for kk in range(KV)]
                    tv = [te[row, pl.ds(tb + kk * L, L)] for kk in range(KV)]
                    tpv = [tp[row, pl.ds(tb + kk * L, L)] for kk in range(KV)]
                    rv = [re_v[row, pl.ds(rb + kk * L, L)] for kk in range(KV)]
                    rpv = [rp_v[row, pl.ds(rb + kk * L, L)] for kk in range(KV)]
                    hdot = hv[0] * hpv[0]
                    tdot = tv[0] * tpv[0]
                    for kk in range(1, KV):
                        hdot = hdot + hv[kk] * hpv[kk]
                        tdot = tdot + tv[kk] * tpv[kk]
                    sh = jnp.sum(hdot)
                    st = jnp.sum(tdot)
                    dacc = None
                    for kk in range(KV):
                        hprime = hv[kk] + sh * rpv[kk]
                        tprime = tv[kk] + st * rpv[kk]
                        he[row, pl.ds(kk * L, L)] = hprime
                        te[row, pl.ds(kk * L, L)] = tprime
                        term = jnp.abs(hprime + rv[kk] - tprime)
                        dacc = term if dacc is None else dacc + term
                    d = jnp.sum(dacc)
                    d_vec = jnp.where(lane == rl, d, d_vec)
                dv[pl.ds(g * L, L)] = d_vec
                return gcarry

            lax.fori_loop(0, C // L, group_body, 0)
            pltpu.sync_copy(he, h_out.at[pl.ds(base, C)])
            pltpu.sync_copy(te, t_out.at[pl.ds(base, C)])
            pltpu.sync_copy(dv, dist_o.at[pl.ds(base, C)])

        stage_and_fire(0, 0)

        def super_step(kk2, carry):
            cc = kk2 * 2
            stage_and_fire(1, cc + 1)
            drain(0)
            compute(0, cc)

            @pl.when(cc + 2 < n_chunks)
            def _():
                stage_and_fire(0, cc + 2)

            drain(1)
            compute(1, cc + 1)
            return carry

        lax.fori_loop(0, n_chunks // 2, super_step, 0)

    return k(h2, hp_, t2, tp_, r2, rp_, ee, ep, rel_e, rel_p)


def kernel(pos_h, pos_t, pos_r, neg_h, neg_t, neg_r,
           ent_emb, rel_emb, ent_proj, rel_proj):
    i32 = jnp.int32
    d_real = ent_emb.shape[1]
    B = pos_h.shape[0]
    e_p = 524288
    r_p = 512

    r_pad = ((0, 2 * r_p - rel_emb.shape[0]), (0, 0))
    ee2 = _retile(ent_emb.T, e_p, 16384)
    ep2 = _retile(ent_proj.T, e_p, 16384)
    re2 = _retile(jnp.pad(rel_emb, r_pad).T, r_p, r_p)
    rp2 = _retile(jnp.pad(rel_proj, r_pad).T, r_p, r_p)

    def split(a, b, p):
        x = jnp.concatenate([a, b]).astype(i32)
        lo = x < p
        return jnp.where(lo, x, x - p), jnp.where(lo, 0, d_real)

    h2, hp_ = split(pos_h, neg_h, e_p)
    t2, tp_ = split(pos_t, neg_t, e_p)
    r2, rp_ = split(pos_r, neg_r, r_p)

    dist, hrows, trows = _dynmap_sc(h2, hp_, t2, tp_, r2, rp_,
                                    ee2, ep2, re2, rp2, d_real)
    return (dist[:B], dist[B:], hrows[:B, :d_real], trows[:B, :d_real],
            hrows[B:, :d_real], trows[B:, :d_real])


# speedup vs baseline: 1.1328x; 1.1328x over previous
import functools

import jax
import jax.numpy as jnp
from jax import lax
from jax.experimental import pallas as pl
from jax.experimental.pallas import tpu as pltpu
from jax.experimental.pallas import tpu_sc as plsc


def _merge_body(t1_ref, t2_ref, o_ref):
    z = jnp.concatenate([t1_ref[...], t2_ref[...]], axis=0)
    o_ref[...] = jnp.transpose(z)


def _merge(t1, t2, bc):
    n = t1.shape[1]
    grid = pl.cdiv(n, bc)
    return pl.pallas_call(
        _merge_body,
        grid=(grid,),
        in_specs=[
            pl.BlockSpec((64, bc), lambda i: (0, i)),
            pl.BlockSpec((64, bc), lambda i: (0, i)),
        ],
        out_specs=pl.BlockSpec((bc, 128), lambda i: (i, 0)),
        out_shape=jax.ShapeDtypeStruct((n, 128), jnp.float32),
    )(t1, t2)


def _dynmap_sc(h2, t2, r2, ent_m, rel_m, d_real):
    B2 = h2.shape[0]
    D = 128
    info = plsc.get_sparse_core_info()
    NC, NS, L = info.num_cores, info.num_subcores, info.num_lanes
    NW = NC * NS
    C = 128
    b_per_w = B2 // NW
    n_chunks = b_per_w // C
    KV = d_real // L

    mesh = plsc.VectorSubcoreMesh(core_axis_name="c", subcore_axis_name="s")
    f32 = jnp.float32
    out_type = (
        jax.ShapeDtypeStruct((B2,), f32),
        jax.ShapeDtypeStruct((B2, D), f32),
        jax.ShapeDtypeStruct((B2, D), f32),
    )

    idx_t = pltpu.VMEM((C,), jnp.int32)
    row_t = pltpu.VMEM((C, D), f32)

    @functools.partial(
        pl.kernel,
        mesh=mesh,
        out_type=out_type,
        compiler_params=pltpu.CompilerParams(needs_layout_passes=False),
        scratch_types=[
            [[idx_t] * 3, [idx_t] * 3],
            [[row_t] * 3, [row_t] * 3],
            [pltpu.VMEM((C,), f32)] * 2,
            [pltpu.SemaphoreType.DMA] * 2,
        ],
    )
    def k(h2r, t2r, r2r, entt, relt,
          dist_o, h_out, t_out, idx_s, row_s, dv_s, sem_s):
        wid = lax.axis_index("s") * NC + lax.axis_index("c")
        lane = lax.iota(jnp.int32, L)
        idx_in = (h2r, t2r, r2r)

        def stage_and_fire(s, cc):
            base = wid * b_per_w + cc * C
            for j in range(3):
                pltpu.sync_copy(idx_in[j].at[pl.ds(base, C)], idx_s[s][j])
            hrow, trow, rrow = row_s[s]
            pltpu.async_copy(entt.at[idx_s[s][0]], hrow, sem_s[s])
            pltpu.async_copy(entt.at[idx_s[s][1]], trow, sem_s[s])
            pltpu.async_copy(relt.at[idx_s[s][2]], rrow, sem_s[s])

        def drain(s):
            for dst in row_s[s]:
                pltpu.make_async_copy(entt.at[pl.ds(0, C)], dst,
                                      sem_s[s]).wait()

        def compute(s, cc):
            base = wid * b_per_w + cc * C
            hrow, trow, rrow = row_s[s]
            dv = dv_s[s]

            def group_body(g, gcarry):
                d_vec = jnp.zeros((L,), f32)
                for rl in range(L):
                    row = g * L + rl
                    hv = [hrow[row, pl.ds(kk * L, L)] for kk in range(KV)]
                    hpv = [hrow[row, pl.ds(64 + kk * L, L)] for kk in range(KV)]
                    tv = [trow[row, pl.ds(kk * L, L)] for kk in range(KV)]
                    tpv = [trow[row, pl.ds(64 + kk * L, L)] for kk in range(KV)]
                    rv = [rrow[row, pl.ds(kk * L, L)] for kk in range(KV)]
                    rpv = [rrow[row, pl.ds(64 + kk * L, L)] for kk in range(KV)]
                    hdot = hv[0] * hpv[0]
                    tdot = tv[0] * tpv[0]
                    for kk in range(1, KV):
                        hdot = hdot + hv[kk] * hpv[kk]
                        tdot = tdot + tv[kk] * tpv[kk]
                    sh = jnp.sum(hdot)
                    st = jnp.sum(tdot)
                    dacc = None
                    for kk in range(KV):
                        hprime = hv[kk] + sh * rpv[kk]
                        tprime = tv[kk] + st * rpv[kk]
                        hrow[row, pl.ds(kk * L, L)] = hprime
                        trow[row, pl.ds(kk * L, L)] = tprime
                        term = jnp.abs(hprime + rv[kk] - tprime)
                        dacc = term if dacc is None else dacc + term
                    d = jnp.sum(dacc)
                    d_vec = jnp.where(lane == rl, d, d_vec)
                dv[pl.ds(g * L, L)] = d_vec
                return gcarry

            lax.fori_loop(0, C // L, group_body, 0)
            pltpu.sync_copy(hrow, h_out.at[pl.ds(base, C)])
            pltpu.sync_copy(trow, t_out.at[pl.ds(base, C)])
            pltpu.sync_copy(dv, dist_o.at[pl.ds(base, C)])

        stage_and_fire(0, 0)

        def super_step(kk2, carry):
            cc = kk2 * 2
            stage_and_fire(1, cc + 1)
            drain(0)
            compute(0, cc)

            @pl.when(cc + 2 < n_chunks)
            def _():
                stage_and_fire(0, cc + 2)

            drain(1)
            compute(1, cc + 1)
            return carry

        lax.fori_loop(0, n_chunks // 2, super_step, 0)

    return k(h2, t2, r2, ent_m, rel_m)


def kernel(pos_h, pos_t, pos_r, neg_h, neg_t, neg_r,
           ent_emb, rel_emb, ent_proj, rel_proj):
    i32 = jnp.int32
    d_real = ent_emb.shape[1]
    B = pos_h.shape[0]
    r_rows = 1024
    r_pad = ((0, r_rows - rel_emb.shape[0]), (0, 0))

    ent_m = _merge(ent_emb.T, ent_proj.T, 16384)
    rel_m = _merge(jnp.pad(rel_emb, r_pad).T, jnp.pad(rel_proj, r_pad).T,
                   r_rows)

    h2 = jnp.concatenate([pos_h, neg_h]).astype(i32)
    t2 = jnp.concatenate([pos_t, neg_t]).astype(i32)
    r2 = jnp.concatenate([pos_r, neg_r]).astype(i32)

    dist, hrows, trows = _dynmap_sc(h2, t2, r2, ent_m, rel_m, d_real)
    return (dist[:B], dist[B:], hrows[:B, :d_real], trows[:B, :d_real],
            hrows[B:, :d_real], trows[B:, :d_real])


# speedup vs baseline: 1.1369x; 1.0036x over previous
import functools

import jax
import jax.numpy as jnp
from jax import lax
from jax.experimental import pallas as pl
from jax.experimental.pallas import tpu as pltpu
from jax.experimental.pallas import tpu_sc as plsc


def _merge_body(t1_ref, t2_ref, o_ref):
    z = jnp.concatenate([t1_ref[...], t2_ref[...]], axis=0)
    o_ref[...] = jnp.transpose(z)


def _merge(t1, t2, bc):
    n = t1.shape[1]
    grid = pl.cdiv(n, bc)
    return pl.pallas_call(
        _merge_body,
        grid=(grid,),
        in_specs=[
            pl.BlockSpec((64, bc), lambda i: (0, i)),
            pl.BlockSpec((64, bc), lambda i: (0, i)),
        ],
        out_specs=pl.BlockSpec((bc, 128), lambda i: (i, 0)),
        out_shape=jax.ShapeDtypeStruct((n, 128), jnp.float32),
    )(t1, t2)


def _dynmap_sc(h2, t2, r2, ent_m, rel_m, d_real):
    B2 = h2.shape[0]
    D = 128
    info = plsc.get_sparse_core_info()
    NC, NS, L = info.num_cores, info.num_subcores, info.num_lanes
    NW = NC * NS
    C = 128
    b_per_w = B2 // NW
    n_chunks = b_per_w // C
    KV = d_real // L

    mesh = plsc.VectorSubcoreMesh(core_axis_name="c", subcore_axis_name="s")
    f32 = jnp.float32
    out_type = (
        jax.ShapeDtypeStruct((B2,), f32),
        jax.ShapeDtypeStruct((B2, D), f32),
        jax.ShapeDtypeStruct((B2, D), f32),
    )

    idx_t = pltpu.VMEM((C,), jnp.int32)
    row_t = pltpu.VMEM((C, D), f32)

    @functools.partial(
        pl.kernel,
        mesh=mesh,
        out_type=out_type,
        compiler_params=pltpu.CompilerParams(needs_layout_passes=False),
        scratch_types=[
            [[idx_t] * 3, [idx_t] * 3],
            [[row_t] * 3, [row_t] * 3],
            [pltpu.VMEM((C,), f32)] * 2,
            [pltpu.SemaphoreType.DMA] * 2,
        ],
    )
    def k(h2r, t2r, r2r, entt, relt,
          dist_o, h_out, t_out, idx_s, row_s, dv_s, sem_s):
        wid = lax.axis_index("s") * NC + lax.axis_index("c")
        lane = lax.iota(jnp.int32, L)
        idx_in = (h2r, t2r, r2r)

        def stage_and_fire(s, cc):
            base = wid * b_per_w + cc * C
            for j in range(3):
                pltpu.sync_copy(idx_in[j].at[pl.ds(base, C)], idx_s[s][j])
            hrow, trow, rrow = row_s[s]
            pltpu.async_copy(entt.at[idx_s[s][0]], hrow, sem_s[s])
            pltpu.async_copy(entt.at[idx_s[s][1]], trow, sem_s[s])
            pltpu.async_copy(relt.at[idx_s[s][2]], rrow, sem_s[s])

        def drain(s):
            for dst in row_s[s]:
                pltpu.make_async_copy(entt.at[pl.ds(0, C)], dst,
                                      sem_s[s]).wait()

        def compute(s, cc):
            base = wid * b_per_w + cc * C
            hrow, trow, rrow = row_s[s]
            dv = dv_s[s]

            def group_body(g, gcarry):
                d_vec = jnp.zeros((L,), f32)
                for rl in range(L):
                    row = g * L + rl
                    hv = [hrow[row, pl.ds(kk * L, L)] for kk in range(KV)]
                    hpv = [hrow[row, pl.ds(64 + kk * L, L)] for kk in range(KV)]
                    tv = [trow[row, pl.ds(kk * L, L)] for kk in range(KV)]
                    tpv = [trow[row, pl.ds(64 + kk * L, L)] for kk in range(KV)]
                    rv = [rrow[row, pl.ds(kk * L, L)] for kk in range(KV)]
                    rpv = [rrow[row, pl.ds(64 + kk * L, L)] for kk in range(KV)]
                    hdot = hv[0] * hpv[0]
                    tdot = tv[0] * tpv[0]
                    for kk in range(1, KV):
                        hdot = hdot + hv[kk] * hpv[kk]
                        tdot = tdot + tv[kk] * tpv[kk]
                    sh = jnp.sum(hdot)
                    st = jnp.sum(tdot)
                    dacc = None
                    for kk in range(KV):
                        hprime = hv[kk] + sh * rpv[kk]
                        tprime = tv[kk] + st * rpv[kk]
                        hrow[row, pl.ds(kk * L, L)] = hprime
                        trow[row, pl.ds(kk * L, L)] = tprime
                        term = jnp.abs(hprime + rv[kk] - tprime)
                        dacc = term if dacc is None else dacc + term
                    d = jnp.sum(dacc)
                    d_vec = jnp.where(lane == rl, d, d_vec)
                dv[pl.ds(g * L, L)] = d_vec
                return gcarry

            lax.fori_loop(0, C // L, group_body, 0)
            pltpu.sync_copy(hrow, h_out.at[pl.ds(base, C)])
            pltpu.sync_copy(trow, t_out.at[pl.ds(base, C)])
            pltpu.sync_copy(dv, dist_o.at[pl.ds(base, C)])

        stage_and_fire(0, 0)

        def super_step(kk2, carry):
            cc = kk2 * 2
            stage_and_fire(1, cc + 1)
            drain(0)
            compute(0, cc)

            @pl.when(cc + 2 < n_chunks)
            def _():
                stage_and_fire(0, cc + 2)

            drain(1)
            compute(1, cc + 1)
            return carry

        lax.fori_loop(0, n_chunks // 2, super_step, 0)

    return k(h2, t2, r2, ent_m, rel_m)


def kernel(pos_h, pos_t, pos_r, neg_h, neg_t, neg_r,
           ent_emb, rel_emb, ent_proj, rel_proj):
    i32 = jnp.int32
    d_real = ent_emb.shape[1]
    B = pos_h.shape[0]
    r_rows = 1024
    r_pad = ((0, r_rows - rel_emb.shape[0]), (0, 0))

    ent_m = _merge(ent_emb.T, ent_proj.T, 24576)
    rel_m = _merge(jnp.pad(rel_emb, r_pad).T, jnp.pad(rel_proj, r_pad).T,
                   r_rows)

    h2 = jnp.concatenate([pos_h, neg_h]).astype(i32)
    t2 = jnp.concatenate([pos_t, neg_t]).astype(i32)
    r2 = jnp.concatenate([pos_r, neg_r]).astype(i32)

    dist, hrows, trows = _dynmap_sc(h2, t2, r2, ent_m, rel_m, d_real)
    return (dist[:B], dist[B:], hrows[:B, :d_real], trows[:B, :d_real],
            hrows[B:, :d_real], trows[B:, :d_real])
